# baseline (speedup 1.0000x reference)
import functools

import jax
import jax.numpy as jnp
from jax import lax
from jax.experimental import pallas as pl
from jax.experimental.pallas import tpu as pltpu
from jax.experimental.pallas import tpu_sc as plsc

N = 10000
E = 320000
H = 128
BN_EPS = 1e-5

NC = 2
NS = 16
NW = NC * NS
CH = 80
NCH = 128
E_PAD = NW * NCH * CH
N_PAD = 10240
ROWS_PT = N_PAD // NS
NBUF = 4
G = 16


def _agg_body(h_hbm, src_hbm, dst_hbm, zeros_hbm, out_hbm,
              acc_sh, src_g, dst_g, b0, b1, b2, b3,
              sem_z, sg0, sg1, sg2, sg3, ss0, ss1, ss2, ss3):
    cid = lax.axis_index("c")
    sid = lax.axis_index("s")
    wid = sid * NC + cid
    crow = wid * NCH
    bufs = (b0, b1, b2, b3)
    sg = (sg0, sg1, sg2, sg3)
    ss = (ss0, ss1, ss2, ss3)

    zcp = pltpu.make_async_copy(
        zeros_hbm.at[pl.ds(sid * ROWS_PT, ROWS_PT)],
        acc_sh.at[pl.ds(sid * ROWS_PT, ROWS_PT)], sem_z)
    zcp.start()

    def load_group(g):
        pltpu.sync_copy(src_hbm.at[pl.ds(crow + g * G, G)], src_g)
        pltpu.sync_copy(dst_hbm.at[pl.ds(crow + g * G, G)], dst_g)

    def gather(k, b):
        return pltpu.make_async_copy(h_hbm.at[src_g.at[k]], bufs[b], sg[b])

    def scatter(k, b):
        return pltpu.make_async_copy(bufs[b], acc_sh.at[dst_g.at[k]], ss[b])

    load_group(0)
    for b in range(NBUF - 1):
        gather(b, b).start()
    zcp.wait()
    plsc.subcore_barrier()

    for g in range(NCH // G):
        if g > 0:
            load_group(g)
            for b in range(NBUF - 1):
                gather(b, b).start()

        def step(j, carry):
            for b in range(NBUF):
                i = j * NBUF + b
                gather(i, b).wait()
                scatter(i, b).start(add=True)
                bp = (b - 1) % NBUF

                @pl.when(i > 0)
                def _():
                    scatter(i - 1, bp).wait()

                @pl.when(i + NBUF - 1 < G)
                def _():
                    gather(i + NBUF - 1, bp).start()
            return carry

        lax.fori_loop(0, G // NBUF, step, 0)
        scatter(G - 1, (G - 1) % NBUF).wait()

    plsc.subcore_barrier()
    pltpu.sync_copy(acc_sh.at[pl.ds(sid * ROWS_PT, ROWS_PT)],
                    out_hbm.at[cid, pl.ds(sid * ROWS_PT, ROWS_PT)])


_agg_call = pl.kernel(
    _agg_body,
    out_type=jax.ShapeDtypeStruct((NC, N_PAD, H), jnp.float32),
    mesh=plsc.VectorSubcoreMesh(core_axis_name="c", subcore_axis_name="s",
                                num_cores=NC, num_subcores=NS),
    scratch_types=[
        pltpu.VMEM_SHARED((N_PAD, H), jnp.float32),
        pltpu.VMEM((G, CH), jnp.int32),
        pltpu.VMEM((G, CH), jnp.int32),
        pltpu.VMEM((CH, H), jnp.float32),
        pltpu.VMEM((CH, H), jnp.float32),
        pltpu.VMEM((CH, H), jnp.float32),
        pltpu.VMEM((CH, H), jnp.float32),
    ] + [pltpu.SemaphoreType.DMA] * 9,
)


BLK = 2000


def _layer_body(h_ref, p0_ref, p1_ref, W1_ref, b1_ref, W2_ref, b2_ref, o_ref):
    z = h_ref[...] + p0_ref[...] + p1_ref[...]
    a = jnp.dot(z, W1_ref[...], preferred_element_type=jnp.float32)
    a = jnp.maximum(a + b1_ref[...], 0.0)
    o = jnp.dot(a, W2_ref[...], preferred_element_type=jnp.float32)
    o_ref[...] = jnp.maximum(o + b2_ref[...], 0.0)


def _mlp_layer(h, p0, p1, W1, b1, W2, b2):
    row = pl.BlockSpec((BLK, H), lambda i: (i, 0))
    full = pl.BlockSpec((H, H), lambda i: (0, 0))
    vec = pl.BlockSpec((1, H), lambda i: (0, 0))
    return pl.pallas_call(
        _layer_body,
        grid=(N // BLK,),
        in_specs=[row, row, row, full, vec, full, vec],
        out_specs=row,
        out_shape=jax.ShapeDtypeStruct((N, H), jnp.float32),
    )(h, p0, p1, W1, b1.reshape(1, H), W2, b2.reshape(1, H))


def _final_body(h1_ref, h2_ref, h3_ref, Wj1_ref, Wj2_ref, Wj3_ref, bjk_ref,
                Wc1_ref, bc1_ref, g_ref, b_ref, m_ref, v_ref, Wc2_ref, bc2_ref,
                o_ref):
    t = jnp.dot(h1_ref[...], Wj1_ref[...], preferred_element_type=jnp.float32)
    t += jnp.dot(h2_ref[...], Wj2_ref[...], preferred_element_type=jnp.float32)
    t += jnp.dot(h3_ref[...], Wj3_ref[...], preferred_element_type=jnp.float32)
    t += bjk_ref[...]
    u = jnp.dot(t, Wc1_ref[...], preferred_element_type=jnp.float32)
    u = u + bc1_ref[...]
    u = (u - m_ref[...]) / jnp.sqrt(v_ref[...] + BN_EPS) * g_ref[...] + b_ref[...]
    u = jnp.maximum(u, 0.0)
    o = jnp.dot(u, Wc2_ref[...], preferred_element_type=jnp.float32)
    o_ref[...] = o + bc2_ref[...]


def _final(h1, h2, h3, W_jk, b_jk, Wc1, bc1, g, b, m, v, Wc2, bc2):
    row = pl.BlockSpec((BLK, H), lambda i: (i, 0))
    full = pl.BlockSpec((H, H), lambda i: (0, 0))
    vec = pl.BlockSpec((1, H), lambda i: (0, 0))
    return pl.pallas_call(
        _final_body,
        grid=(N // BLK,),
        in_specs=[row, row, row, full, full, full, vec, full, vec,
                  vec, vec, vec, vec, full, vec],
        out_specs=row,
        out_shape=jax.ShapeDtypeStruct((N, H), jnp.float32),
    )(h1, h2, h3, W_jk[0:H], W_jk[H:2 * H], W_jk[2 * H:3 * H],
      b_jk.reshape(1, H), Wc1, bc1.reshape(1, H), g.reshape(1, H),
      b.reshape(1, H), m.reshape(1, H), v.reshape(1, H), Wc2,
      bc2.reshape(1, H))


def kernel(x, edge_index, W1_0, b1_0, W2_0, b2_0, W1_1, b1_1, W2_1, b2_1,
           W1_2, b1_2, W2_2, b2_2, W_jk, b_jk, Wc1, bc1, bn_gamma, bn_beta,
           bn_mean, bn_var, Wc2, bc2):
    pad = jnp.arange(E_PAD - E, dtype=jnp.int32)
    src = jnp.concatenate([edge_index[0], pad % N]).reshape(NW * NCH, CH)
    dst = jnp.concatenate([edge_index[1], N + pad % (N_PAD - N)]
                          ).reshape(NW * NCH, CH)
    zeros = jnp.zeros((N_PAD, H), jnp.float32)
    Ws = [(W1_0, b1_0, W2_0, b2_0), (W1_1, b1_1, W2_1, b2_1),
          (W1_2, b1_2, W2_2, b2_2)]
    h = x
    xs = []
    for (W1, b1, W2, b2) in Ws:
        parts = _agg_call(h, src, dst, zeros)
        h = _mlp_layer(h, parts[0, :N], parts[1, :N], W1, b1, W2, b2)
        xs.append(h)
    return _final(xs[0], xs[1], xs[2], W_jk, b_jk, Wc1, bc1, bn_gamma,
                  bn_beta, bn_mean, bn_var, Wc2, bc2)


# speedup vs baseline: 1.0071x; 1.0071x over previous
import functools

import jax
import jax.numpy as jnp
from jax import lax
from jax.experimental import pallas as pl
from jax.experimental.pallas import tpu as pltpu
from jax.experimental.pallas import tpu_sc as plsc

N = 10000
E = 320000
H = 128
BN_EPS = 1e-5

NC = 2
NS = 16
NW = NC * NS
CH = 128
NCH = 80
E_PAD = NW * NCH * CH
N_PAD = 10240
ROWS_PT = N_PAD // NS
NBUF = 2
G = 40


def _agg_body(h_hbm, src_hbm, dst_hbm, zeros_hbm, out_hbm,
              acc_sh, src_g, dst_g, b0, b1,
              sem_z, sg0, sg1, ss0, ss1):
    cid = lax.axis_index("c")
    sid = lax.axis_index("s")
    wid = sid * NC + cid
    crow = wid * NCH
    bufs = (b0, b1)
    sg = (sg0, sg1)
    ss = (ss0, ss1)

    zcp = pltpu.make_async_copy(
        zeros_hbm.at[pl.ds(sid * ROWS_PT, ROWS_PT)],
        acc_sh.at[pl.ds(sid * ROWS_PT, ROWS_PT)], sem_z)
    zcp.start()

    def load_group(g):
        pltpu.sync_copy(src_hbm.at[pl.ds(crow + g * G, G)], src_g)
        pltpu.sync_copy(dst_hbm.at[pl.ds(crow + g * G, G)], dst_g)

    def gather(k, b):
        return pltpu.make_async_copy(h_hbm.at[src_g.at[k]], bufs[b], sg[b])

    def scatter(k, b):
        return pltpu.make_async_copy(bufs[b], acc_sh.at[dst_g.at[k]], ss[b])

    load_group(0)
    for b in range(NBUF - 1):
        gather(b, b).start()
    zcp.wait()
    plsc.subcore_barrier()

    for g in range(NCH // G):
        if g > 0:
            load_group(g)
            for b in range(NBUF - 1):
                gather(b, b).start()

        def step(j, carry):
            for b in range(NBUF):
                i = j * NBUF + b
                gather(i, b).wait()
                scatter(i, b).start(add=True)
                bp = (b - 1) % NBUF

                @pl.when(i > 0)
                def _():
                    scatter(i - 1, bp).wait()

                @pl.when(i + NBUF - 1 < G)
                def _():
                    gather(i + NBUF - 1, bp).start()
            return carry

        lax.fori_loop(0, G // NBUF, step, 0)
        scatter(G - 1, (G - 1) % NBUF).wait()

    plsc.subcore_barrier()
    pltpu.sync_copy(acc_sh.at[pl.ds(sid * ROWS_PT, ROWS_PT)],
                    out_hbm.at[cid, pl.ds(sid * ROWS_PT, ROWS_PT)])


_agg_call = pl.kernel(
    _agg_body,
    out_type=jax.ShapeDtypeStruct((NC, N_PAD, H), jnp.float32),
    mesh=plsc.VectorSubcoreMesh(core_axis_name="c", subcore_axis_name="s",
                                num_cores=NC, num_subcores=NS),
    scratch_types=[
        pltpu.VMEM_SHARED((N_PAD, H), jnp.float32),
        pltpu.VMEM((G, CH), jnp.int32),
        pltpu.VMEM((G, CH), jnp.int32),
        pltpu.VMEM((CH, H), jnp.float32),
        pltpu.VMEM((CH, H), jnp.float32),
    ] + [pltpu.SemaphoreType.DMA] * 5,
)


BLK = 2000


def _mlp_block(h, parts, W1, b1, W2, b2):
    z = h + parts[0] + parts[1]
    a = jnp.dot(z, W1, preferred_element_type=jnp.float32)
    a = jnp.maximum(a + b1, 0.0)
    o = jnp.dot(a, W2, preferred_element_type=jnp.float32)
    return jnp.maximum(o + b2, 0.0)


def _layer_body(h_ref, p_ref, W1_ref, b1_ref, W2_ref, b2_ref, o_ref):
    o_ref[...] = _mlp_block(h_ref[...], p_ref[...], W1_ref[...], b1_ref[...],
                            W2_ref[...], b2_ref[...])


def _mlp_layer(h, parts, W1, b1, W2, b2):
    row = pl.BlockSpec((BLK, H), lambda i: (i, 0))
    prt = pl.BlockSpec((NC, BLK, H), lambda i: (0, i, 0))
    full = pl.BlockSpec((H, H), lambda i: (0, 0))
    vec = pl.BlockSpec((1, H), lambda i: (0, 0))
    return pl.pallas_call(
        _layer_body,
        grid=(N // BLK,),
        in_specs=[row, prt, full, vec, full, vec],
        out_specs=row,
        out_shape=jax.ShapeDtypeStruct((N, H), jnp.float32),
    )(h, parts, W1, b1.reshape(1, H), W2, b2.reshape(1, H))


def _final_body(h1_ref, h2_ref, p_ref, W1_ref, b1_ref, W2_ref, b2_ref,
                Wj1_ref, Wj2_ref, Wj3_ref, bjk_ref,
                Wc1_ref, bc1_ref, g_ref, b_ref, m_ref, v_ref, Wc2_ref, bc2_ref,
                o_ref):
    h3 = _mlp_block(h2_ref[...], p_ref[...], W1_ref[...], b1_ref[...],
                    W2_ref[...], b2_ref[...])
    t = jnp.dot(h1_ref[...], Wj1_ref[...], preferred_element_type=jnp.float32)
    t += jnp.dot(h2_ref[...], Wj2_ref[...], preferred_element_type=jnp.float32)
    t += jnp.dot(h3, Wj3_ref[...], preferred_element_type=jnp.float32)
    t += bjk_ref[...]
    u = jnp.dot(t, Wc1_ref[...], preferred_element_type=jnp.float32)
    u = u + bc1_ref[...]
    u = (u - m_ref[...]) / jnp.sqrt(v_ref[...] + BN_EPS) * g_ref[...] + b_ref[...]
    u = jnp.maximum(u, 0.0)
    o = jnp.dot(u, Wc2_ref[...], preferred_element_type=jnp.float32)
    o_ref[...] = o + bc2_ref[...]


def _final(h1, h2, parts3, W1, b1, W2, b2,
           W_jk, b_jk, Wc1, bc1, g, b, m, v, Wc2, bc2):
    row = pl.BlockSpec((BLK, H), lambda i: (i, 0))
    prt = pl.BlockSpec((NC, BLK, H), lambda i: (0, i, 0))
    full = pl.BlockSpec((H, H), lambda i: (0, 0))
    vec = pl.BlockSpec((1, H), lambda i: (0, 0))
    return pl.pallas_call(
        _final_body,
        grid=(N // BLK,),
        in_specs=[row, row, prt, full, vec, full, vec,
                  full, full, full, vec, full, vec,
                  vec, vec, vec, vec, full, vec],
        out_specs=row,
        out_shape=jax.ShapeDtypeStruct((N, H), jnp.float32),
    )(h1, h2, parts3, W1, b1.reshape(1, H), W2, b2.reshape(1, H),
      W_jk[0:H], W_jk[H:2 * H], W_jk[2 * H:3 * H],
      b_jk.reshape(1, H), Wc1, bc1.reshape(1, H), g.reshape(1, H),
      b.reshape(1, H), m.reshape(1, H), v.reshape(1, H), Wc2,
      bc2.reshape(1, H))


def kernel(x, edge_index, W1_0, b1_0, W2_0, b2_0, W1_1, b1_1, W2_1, b2_1,
           W1_2, b1_2, W2_2, b2_2, W_jk, b_jk, Wc1, bc1, bn_gamma, bn_beta,
           bn_mean, bn_var, Wc2, bc2):
    pad = jnp.arange(E_PAD - E, dtype=jnp.int32)
    src = jnp.concatenate([edge_index[0], pad % N]).reshape(NW * NCH, CH)
    dst = jnp.concatenate([edge_index[1], N + pad % (N_PAD - N)]
                          ).reshape(NW * NCH, CH)
    zeros = jnp.zeros((N_PAD, H), jnp.float32)
    parts = _agg_call(x, src, dst, zeros)
    h1 = _mlp_layer(x, parts, W1_0, b1_0, W2_0, b2_0)
    parts = _agg_call(h1, src, dst, zeros)
    h2 = _mlp_layer(h1, parts, W1_1, b1_1, W2_1, b2_1)
    parts3 = _agg_call(h2, src, dst, zeros)
    return _final(h1, h2, parts3, W1_2, b1_2, W2_2, b2_2,
                  W_jk, b_jk, Wc1, bc1, bn_gamma, bn_beta, bn_mean, bn_var,
                  Wc2, bc2)


# speedup vs baseline: 1.1777x; 1.1694x over previous
import functools

import jax
import jax.numpy as jnp
from jax import lax
from jax.experimental import pallas as pl
from jax.experimental.pallas import tpu as pltpu
from jax.experimental.pallas import tpu_sc as plsc

N = 10000
E = 320000
H = 128
BN_EPS = 1e-5

NC = 2
NS = 16
NW = NC * NS
CH = 128
NCH = 80
E_PAD = NW * NCH * CH
N_PAD = 10240
ROWS_PT = N_PAD // NS
NBUF = 2
G = 40


def _agg_body(h_hbm, src_hbm, dst_hbm, zeros_hbm, out_hbm,
              acc_sh, src_g, dst_g, b0, b1,
              sem_z, sg0, sg1, ss0, ss1):
    cid = lax.axis_index("c")
    sid = lax.axis_index("s")
    wid = sid * NC + cid
    crow = wid * NCH
    bufs = (b0, b1)
    sg = (sg0, sg1)
    ss = (ss0, ss1)

    zcp = pltpu.make_async_copy(
        zeros_hbm.at[pl.ds(sid * ROWS_PT, ROWS_PT)],
        acc_sh.at[pl.ds(sid * ROWS_PT, ROWS_PT)], sem_z)
    zcp.start()

    def load_group(g):
        pltpu.sync_copy(src_hbm.at[pl.ds(crow + g * G, G)], src_g)
        pltpu.sync_copy(dst_hbm.at[pl.ds(crow + g * G, G)], dst_g)

    def gather(k, b):
        return pltpu.make_async_copy(h_hbm.at[src_g.at[k]], bufs[b], sg[b])

    def scatter(k, b):
        return pltpu.make_async_copy(bufs[b], acc_sh.at[dst_g.at[k]], ss[b])

    load_group(0)
    for b in range(NBUF):
        gather(b, b).start()
    zcp.wait()
    plsc.subcore_barrier()

    for g in range(NCH // G):
        if g > 0:
            load_group(g)
            for b in range(NBUF):
                gather(b, b).start()

        def step(j, carry):
            for b in range(NBUF):
                i = j * NBUF + b
                gather(i, b).wait()
                pltpu.sync_copy(bufs[b], acc_sh.at[dst_g.at[i]], add=True)

                @pl.when(i + NBUF < G)
                def _():
                    gather(i + NBUF, b).start()
            return carry

        lax.fori_loop(0, G // NBUF, step, 0)

    plsc.subcore_barrier()
    pltpu.sync_copy(acc_sh.at[pl.ds(sid * ROWS_PT, ROWS_PT)],
                    out_hbm.at[cid, pl.ds(sid * ROWS_PT, ROWS_PT)])


_agg_call = pl.kernel(
    _agg_body,
    out_type=jax.ShapeDtypeStruct((NC, N_PAD, H), jnp.float32),
    mesh=plsc.VectorSubcoreMesh(core_axis_name="c", subcore_axis_name="s",
                                num_cores=NC, num_subcores=NS),
    scratch_types=[
        pltpu.VMEM_SHARED((N_PAD, H), jnp.float32),
        pltpu.VMEM((G, CH), jnp.int32),
        pltpu.VMEM((G, CH), jnp.int32),
        pltpu.VMEM((CH, H), jnp.float32),
        pltpu.VMEM((CH, H), jnp.float32),
    ] + [pltpu.SemaphoreType.DMA] * 5,
)


BLK = 2000


def _mlp_block(h, parts, W1, b1, W2, b2):
    z = h + parts[0] + parts[1]
    a = jnp.dot(z, W1, preferred_element_type=jnp.float32)
    a = jnp.maximum(a + b1, 0.0)
    o = jnp.dot(a, W2, preferred_element_type=jnp.float32)
    return jnp.maximum(o + b2, 0.0)


def _layer_body(h_ref, p_ref, W1_ref, b1_ref, W2_ref, b2_ref, o_ref):
    o_ref[...] = _mlp_block(h_ref[...], p_ref[...], W1_ref[...], b1_ref[...],
                            W2_ref[...], b2_ref[...])


def _mlp_layer(h, parts, W1, b1, W2, b2):
    row = pl.BlockSpec((BLK, H), lambda i: (i, 0))
    prt = pl.BlockSpec((NC, BLK, H), lambda i: (0, i, 0))
    full = pl.BlockSpec((H, H), lambda i: (0, 0))
    vec = pl.BlockSpec((1, H), lambda i: (0, 0))
    return pl.pallas_call(
        _layer_body,
        grid=(N // BLK,),
        in_specs=[row, prt, full, vec, full, vec],
        out_specs=row,
        out_shape=jax.ShapeDtypeStruct((N, H), jnp.float32),
    )(h, parts, W1, b1.reshape(1, H), W2, b2.reshape(1, H))


def _final_body(h1_ref, h2_ref, p_ref, W1_ref, b1_ref, W2_ref, b2_ref,
                Wj1_ref, Wj2_ref, Wj3_ref, bjk_ref,
                Wc1_ref, bc1_ref, g_ref, b_ref, m_ref, v_ref, Wc2_ref, bc2_ref,
                o_ref):
    h3 = _mlp_block(h2_ref[...], p_ref[...], W1_ref[...], b1_ref[...],
                    W2_ref[...], b2_ref[...])
    t = jnp.dot(h1_ref[...], Wj1_ref[...], preferred_element_type=jnp.float32)
    t += jnp.dot(h2_ref[...], Wj2_ref[...], preferred_element_type=jnp.float32)
    t += jnp.dot(h3, Wj3_ref[...], preferred_element_type=jnp.float32)
    t += bjk_ref[...]
    u = jnp.dot(t, Wc1_ref[...], preferred_element_type=jnp.float32)
    u = u + bc1_ref[...]
    u = (u - m_ref[...]) / jnp.sqrt(v_ref[...] + BN_EPS) * g_ref[...] + b_ref[...]
    u = jnp.maximum(u, 0.0)
    o = jnp.dot(u, Wc2_ref[...], preferred_element_type=jnp.float32)
    o_ref[...] = o + bc2_ref[...]


def _final(h1, h2, parts3, W1, b1, W2, b2,
           W_jk, b_jk, Wc1, bc1, g, b, m, v, Wc2, bc2):
    row = pl.BlockSpec((BLK, H), lambda i: (i, 0))
    prt = pl.BlockSpec((NC, BLK, H), lambda i: (0, i, 0))
    full = pl.BlockSpec((H, H), lambda i: (0, 0))
    vec = pl.BlockSpec((1, H), lambda i: (0, 0))
    return pl.pallas_call(
        _final_body,
        grid=(N // BLK,),
        in_specs=[row, row, prt, full, vec, full, vec,
                  full, full, full, vec, full, vec,
                  vec, vec, vec, vec, full, vec],
        out_specs=row,
        out_shape=jax.ShapeDtypeStruct((N, H), jnp.float32),
    )(h1, h2, parts3, W1, b1.reshape(1, H), W2, b2.reshape(1, H),
      W_jk[0:H], W_jk[H:2 * H], W_jk[2 * H:3 * H],
      b_jk.reshape(1, H), Wc1, bc1.reshape(1, H), g.reshape(1, H),
      b.reshape(1, H), m.reshape(1, H), v.reshape(1, H), Wc2,
      bc2.reshape(1, H))


def kernel(x, edge_index, W1_0, b1_0, W2_0, b2_0, W1_1, b1_1, W2_1, b2_1,
           W1_2, b1_2, W2_2, b2_2, W_jk, b_jk, Wc1, bc1, bn_gamma, bn_beta,
           bn_mean, bn_var, Wc2, bc2):
    pad = jnp.arange(E_PAD - E, dtype=jnp.int32)
    src = jnp.concatenate([edge_index[0], pad % N]).reshape(NW * NCH, CH)
    dst = jnp.concatenate([edge_index[1], N + pad % (N_PAD - N)]
                          ).reshape(NW * NCH, CH)
    zeros = jnp.zeros((N_PAD, H), jnp.float32)
    parts = _agg_call(x, src, dst, zeros)
    h1 = _mlp_layer(x, parts, W1_0, b1_0, W2_0, b2_0)
    parts = _agg_call(h1, src, dst, zeros)
    h2 = _mlp_layer(h1, parts, W1_1, b1_1, W2_1, b2_1)
    parts3 = _agg_call(h2, src, dst, zeros)
    return _final(h1, h2, parts3, W1_2, b1_2, W2_2, b2_2,
                  W_jk, b_jk, Wc1, bc1, bn_gamma, bn_beta, bn_mean, bn_var,
                  Wc2, bc2)


# speedup vs baseline: 1.1792x; 1.0012x over previous
import functools

import jax
import jax.numpy as jnp
from jax import lax
from jax.experimental import pallas as pl
from jax.experimental.pallas import tpu as pltpu
from jax.experimental.pallas import tpu_sc as plsc

N = 10000
E = 320000
H = 128
BN_EPS = 1e-5

NC = 2
NS = 16
NW = NC * NS
CH = 128
NCH = 80
E_PAD = NW * NCH * CH
N_PAD = 10240
ROWS_PT = N_PAD // NS
NBUF = 2
G = 40


def _agg_body(h_hbm, src_hbm, dst_hbm, zeros_hbm, out_hbm,
              acc_sh, src_g, dst_g, b0, b1,
              sem_z, sg0, sg1, ss0, ss1):
    cid = lax.axis_index("c")
    sid = lax.axis_index("s")
    wid = sid * NC + cid
    crow = wid * NCH
    bufs = (b0, b1)
    sg = (sg0, sg1)
    ss = (ss0, ss1)

    zcp = pltpu.make_async_copy(
        zeros_hbm.at[pl.ds(sid * ROWS_PT, ROWS_PT)],
        acc_sh.at[pl.ds(sid * ROWS_PT, ROWS_PT)], sem_z)
    zcp.start()

    def load_group(g):
        pltpu.sync_copy(src_hbm.at[pl.ds(crow + g * G, G)], src_g)
        pltpu.sync_copy(dst_hbm.at[pl.ds(crow + g * G, G)], dst_g)

    HC = CH // 2

    def gather_half(k, b, half):
        return pltpu.make_async_copy(
            h_hbm.at[src_g.at[k, pl.ds(half * HC, HC)]],
            bufs[b].at[pl.ds(half * HC, HC)], sg[b])

    class _Gather:
        def __init__(self, k, b):
            self.k, self.b = k, b

        def start(self):
            gather_half(self.k, self.b, 0).start()
            gather_half(self.k, self.b, 1).start()

        def wait(self):
            gather_half(self.k, self.b, 0).wait()
            gather_half(self.k, self.b, 1).wait()

    def gather(k, b):
        return _Gather(k, b)

    def scatter(k, b):
        return pltpu.make_async_copy(bufs[b], acc_sh.at[dst_g.at[k]], ss[b])

    load_group(0)
    for b in range(NBUF):
        gather(b, b).start()
    zcp.wait()
    plsc.subcore_barrier()

    for g in range(NCH // G):
        if g > 0:
            load_group(g)
            for b in range(NBUF):
                gather(b, b).start()

        def step(j, carry):
            for b in range(NBUF):
                i = j * NBUF + b
                gather(i, b).wait()
                pltpu.sync_copy(bufs[b], acc_sh.at[dst_g.at[i]], add=True)

                @pl.when(i + NBUF < G)
                def _():
                    gather(i + NBUF, b).start()
            return carry

        lax.fori_loop(0, G // NBUF, step, 0)

    plsc.subcore_barrier()
    pltpu.sync_copy(acc_sh.at[pl.ds(sid * ROWS_PT, ROWS_PT)],
                    out_hbm.at[cid, pl.ds(sid * ROWS_PT, ROWS_PT)])


_agg_call = pl.kernel(
    _agg_body,
    out_type=jax.ShapeDtypeStruct((NC, N_PAD, H), jnp.float32),
    mesh=plsc.VectorSubcoreMesh(core_axis_name="c", subcore_axis_name="s",
                                num_cores=NC, num_subcores=NS),
    scratch_types=[
        pltpu.VMEM_SHARED((N_PAD, H), jnp.float32),
        pltpu.VMEM((G, CH), jnp.int32),
        pltpu.VMEM((G, CH), jnp.int32),
        pltpu.VMEM((CH, H), jnp.float32),
        pltpu.VMEM((CH, H), jnp.float32),
    ] + [pltpu.SemaphoreType.DMA] * 5,
)


BLK = 2000


def _mlp_block(h, parts, W1, b1, W2, b2):
    z = h + parts[0] + parts[1]
    a = jnp.dot(z, W1, preferred_element_type=jnp.float32)
    a = jnp.maximum(a + b1, 0.0)
    o = jnp.dot(a, W2, preferred_element_type=jnp.float32)
    return jnp.maximum(o + b2, 0.0)


def _layer_body(h_ref, p_ref, W1_ref, b1_ref, W2_ref, b2_ref, o_ref):
    o_ref[...] = _mlp_block(h_ref[...], p_ref[...], W1_ref[...], b1_ref[...],
                            W2_ref[...], b2_ref[...])


def _mlp_layer(h, parts, W1, b1, W2, b2):
    row = pl.BlockSpec((BLK, H), lambda i: (i, 0))
    prt = pl.BlockSpec((NC, BLK, H), lambda i: (0, i, 0))
    full = pl.BlockSpec((H, H), lambda i: (0, 0))
    vec = pl.BlockSpec((1, H), lambda i: (0, 0))
    return pl.pallas_call(
        _layer_body,
        grid=(N // BLK,),
        in_specs=[row, prt, full, vec, full, vec],
        out_specs=row,
        out_shape=jax.ShapeDtypeStruct((N, H), jnp.float32),
    )(h, parts, W1, b1.reshape(1, H), W2, b2.reshape(1, H))


def _final_body(h1_ref, h2_ref, p_ref, W1_ref, b1_ref, W2_ref, b2_ref,
                Wj1_ref, Wj2_ref, Wj3_ref, bjk_ref,
                Wc1_ref, bc1_ref, g_ref, b_ref, m_ref, v_ref, Wc2_ref, bc2_ref,
                o_ref):
    h3 = _mlp_block(h2_ref[...], p_ref[...], W1_ref[...], b1_ref[...],
                    W2_ref[...], b2_ref[...])
    t = jnp.dot(h1_ref[...], Wj1_ref[...], preferred_element_type=jnp.float32)
    t += jnp.dot(h2_ref[...], Wj2_ref[...], preferred_element_type=jnp.float32)
    t += jnp.dot(h3, Wj3_ref[...], preferred_element_type=jnp.float32)
    t += bjk_ref[...]
    u = jnp.dot(t, Wc1_ref[...], preferred_element_type=jnp.float32)
    u = u + bc1_ref[...]
    u = (u - m_ref[...]) / jnp.sqrt(v_ref[...] + BN_EPS) * g_ref[...] + b_ref[...]
    u = jnp.maximum(u, 0.0)
    o = jnp.dot(u, Wc2_ref[...], preferred_element_type=jnp.float32)
    o_ref[...] = o + bc2_ref[...]


def _final(h1, h2, parts3, W1, b1, W2, b2,
           W_jk, b_jk, Wc1, bc1, g, b, m, v, Wc2, bc2):
    row = pl.BlockSpec((BLK, H), lambda i: (i, 0))
    prt = pl.BlockSpec((NC, BLK, H), lambda i: (0, i, 0))
    full = pl.BlockSpec((H, H), lambda i: (0, 0))
    vec = pl.BlockSpec((1, H), lambda i: (0, 0))
    return pl.pallas_call(
        _final_body,
        grid=(N // BLK,),
        in_specs=[row, row, prt, full, vec, full, vec,
                  full, full, full, vec, full, vec,
                  vec, vec, vec, vec, full, vec],
        out_specs=row,
        out_shape=jax.ShapeDtypeStruct((N, H), jnp.float32),
    )(h1, h2, parts3, W1, b1.reshape(1, H), W2, b2.reshape(1, H),
      W_jk[0:H], W_jk[H:2 * H], W_jk[2 * H:3 * H],
      b_jk.reshape(1, H), Wc1, bc1.reshape(1, H), g.reshape(1, H),
      b.reshape(1, H), m.reshape(1, H), v.reshape(1, H), Wc2,
      bc2.reshape(1, H))


def kernel(x, edge_index, W1_0, b1_0, W2_0, b2_0, W1_1, b1_1, W2_1, b2_1,
           W1_2, b1_2, W2_2, b2_2, W_jk, b_jk, Wc1, bc1, bn_gamma, bn_beta,
           bn_mean, bn_var, Wc2, bc2):
    pad = jnp.arange(E_PAD - E, dtype=jnp.int32)
    src = jnp.concatenate([edge_index[0], pad % N]).reshape(NW * NCH, CH)
    dst = jnp.concatenate([edge_index[1], N + pad % (N_PAD - N)]
                          ).reshape(NW * NCH, CH)
    zeros = jnp.zeros((N_PAD, H), jnp.float32)
    parts = _agg_call(x, src, dst, zeros)
    h1 = _mlp_layer(x, parts, W1_0, b1_0, W2_0, b2_0)
    parts = _agg_call(h1, src, dst, zeros)
    h2 = _mlp_layer(h1, parts, W1_1, b1_1, W2_1, b2_1)
    parts3 = _agg_call(h2, src, dst, zeros)
    return _final(h1, h2, parts3, W1_2, b1_2, W2_2, b2_2,
                  W_jk, b_jk, Wc1, bc1, bn_gamma, bn_beta, bn_mean, bn_var,
                  Wc2, bc2)
